# baseline (speedup 1.0000x reference)
import jax
import jax.numpy as jnp
from jax.experimental import pallas as pl

DIM = 384
N = 4096
CORR_DIM = 882
G_KK_C = 512
G_IJ_C = 64
BLK = 256
NBLK = N // BLK

f32 = jnp.float32
i32 = jnp.int32


def _dgT(x, w):
    return jax.lax.dot_general(
        x, w, dimension_numbers=(((1,), (1,)), ((), ())),
        preferred_element_type=f32)


def _dg(x, w):
    return jax.lax.dot_general(
        x, w, dimension_numbers=(((1,), (0,)), ((), ())),
        preferred_element_type=f32)


def _dgTT(x, w):
    return jax.lax.dot_general(
        x, w, dimension_numbers=(((0,), (0,)), ((), ())),
        preferred_element_type=f32)


def _dg_hi(x, w):
    return jax.lax.dot_general(
        x, w, dimension_numbers=(((1,), (0,)), ((), ())),
        preferred_element_type=f32, precision=jax.lax.Precision.HIGHEST)


def _dgTT_hi(x, w):
    return jax.lax.dot_general(
        x, w, dimension_numbers=(((0,), (0,)), ((), ())),
        preferred_element_type=f32, precision=jax.lax.Precision.HIGHEST)


def _ln(x, g, b, eps=1e-3):
    mu = jnp.mean(x, axis=-1, keepdims=True)
    var = jnp.mean((x - mu) ** 2, axis=-1, keepdims=True)
    return (x - mu) / jnp.sqrt(var + eps) * g + b


def _k1(corr_ref, net_ref, inp_ref, ii_ref,
        w1, b1, w2, b2, lng, lnb, w3, b3, ng, nb, out_ref):
    c = jax.nn.relu(_dgT(corr_ref[...], w1[...]) + b1[...])
    c = _dgT(c, w2[...]) + b2[...]
    c = _ln(c, lng[...], lnb[...])
    c = jax.nn.relu(c)
    c = _dgT(c, w3[...]) + b3[...]
    ii_bias = jnp.sum(ii_ref[...]) * 1e-10
    x = net_ref[...] + inp_ref[...] + c + ii_bias
    out_ref[...] = _ln(x, ng[...], nb[...])


def _nb_tables(kk_row, kk_col, jj_row, jj_col):

    ohkk = (jax.lax.broadcasted_iota(i32, (N, G_KK_C), 1) == kk_col
            ).astype(f32)
    ohjj = (jax.lax.broadcasted_iota(i32, (N, G_IJ_C), 1) == jj_col
            ).astype(f32)

    iota_loc = jax.lax.broadcasted_iota(i32, (BLK, BLK), 1)
    row_loc = jax.lax.broadcasted_iota(i32, (BLK, BLK), 0)
    tri = iota_loc < row_loc
    p_run = jnp.zeros((G_KK_C, G_IJ_C), f32)
    firsts = []
    for b in range(NBLK):
        sl = slice(b * BLK, (b + 1) * BLK)
        okb = ohkk[sl]
        ojb = ohjj[sl]
        cnt_prior = jnp.sum(_dg(okb, p_run) * ojb, axis=1, keepdims=True)
        m2 = ((kk_row[:, sl] == kk_col[sl]) & (jj_row[:, sl] == jj_col[sl])
              & tri)
        cnt_within = jnp.sum(jnp.where(m2, 1.0, 0.0), axis=1, keepdims=True)
        firsts.append(cnt_prior + cnt_within < 0.5)
        p_run = p_run + _dgTT(okb, ojb)
    first = jnp.concatenate(firsts, axis=0).astype(f32)

    idx_i = jax.lax.broadcasted_iota(i32, (N, 1), 0)
    hi_f = (idx_i // G_IJ_C).astype(f32)
    lo_f = (idx_i % G_IJ_C).astype(f32)
    m_hi = _dgTT(ohkk * (hi_f * first), ohjj)
    m_lo = _dgTT(ohkk * (lo_f * first), ohjj)
    row_p = _dg(ohkk, p_run)
    row_hi = _dg(ohkk, m_hi)
    row_lo = _dg(ohkk, m_lo)

    iota64 = jax.lax.broadcasted_iota(i32, (N, G_IJ_C), 1)
    exists = row_p > 0.5
    vstar = jnp.max(jnp.where(exists & (iota64 < jj_col), iota64, -1),
                    axis=1, keepdims=True)
    vprime = jnp.min(jnp.where(exists & (iota64 > jj_col), iota64, G_IJ_C),
                     axis=1, keepdims=True)
    sel_i = (iota64 == vstar)
    ixv = (jnp.sum(jnp.where(sel_i, row_hi, 0.0), axis=1, keepdims=True)
           * G_IJ_C
           + jnp.sum(jnp.where(sel_i, row_lo, 0.0), axis=1, keepdims=True))
    sel_j = (iota64 == vprime)
    jxv = (jnp.sum(jnp.where(sel_j, row_hi, 0.0), axis=1, keepdims=True)
           * G_IJ_C
           + jnp.sum(jnp.where(sel_j, row_lo, 0.0), axis=1, keepdims=True))
    ix = jnp.where(vstar >= 1, ixv.astype(i32), 0)
    jx = jnp.where(vprime < G_IJ_C, jxv.astype(i32), 0)
    return ix, jx


def _gather_mlp(net_in, idx, w1, b1, w2, b2, blk=2048):
    iota = jax.lax.broadcasted_iota(i32, (blk, N), 1)
    blocks = []
    for b in range(N // blk):
        sl = slice(b * blk, (b + 1) * blk)
        oh = (iota == idx[sl]).astype(f32)
        gath = _dg(oh, net_in)
        h = jax.nn.relu(_dgT(gath, w1) + b1)
        upd = _dgT(h, w2) + b2
        blocks.append(net_in[sl] + upd)
    return jnp.concatenate(blocks, axis=0)


def _k2(net_ref, kk_row_ref, kk_col_ref, jj_row_ref, jj_col_ref,
        w1, b1, w2, b2, out_ref, jx_ref):
    ix, jx = _nb_tables(kk_row_ref[...], kk_col_ref[...],
                        jj_row_ref[...], jj_col_ref[...])
    jx_ref[...] = jx
    out_ref[...] = _gather_mlp(net_ref[...], ix,
                               w1[...], b1[...], w2[...], b2[...])


def _soft_agg(x, idx_col, G, fw, fb, gw, gb, hw, hb):
    fx = _dgT(x, fw) + fb
    gx = _dgT(x, gw) + gb
    gmax = jnp.max(gx, axis=0, keepdims=True)
    ex = jnp.exp(gx - gmax)
    oh = (jax.lax.broadcasted_iota(i32, (N, G), 1) == idx_col).astype(f32)
    esum = _dgTT(oh, ex)
    ynum = _dgTT(oh, fx * ex)
    y = ynum / jnp.where(esum > 0, esum, 1.0)
    hy = _dgT(y, hw) + hb
    return _dg(oh, hy)


def _gr(x, gw, gb, r1w, r1b, r2w, r2b):
    gate = jax.nn.sigmoid(_dgT(x, gw) + gb)
    res = _dgT(jax.nn.relu(_dgT(x, r1w) + r1b), r2w) + r2b
    return x + gate * res


def _k_cmlp(net_ref, idx_ref, w1, b1, w2, b2, out_ref):
    out_ref[...] = _gather_mlp(net_ref[...], idx_ref[...],
                               w1[...], b1[...], w2[...], b2[...])


def _k3(x_ref, kkidx_ref, ijidx_ref, ii_ref,
        akfw, akfb, akgw, akgb, akhw, akhb,
        aifw, aifb, aigw, aigb, aihw, aihb,
        l1g, l1b, g1gw, g1gb, g1r1w, g1r1b, g1r2w, g1r2b,
        l2g, l2b, g2gw, g2gb, g2r1w, g2r1b, g2r2w, g2r2b,
        wdw, bdw, out_net_ref, out_dw_ref):
    x = x_ref[...]
    x = x + _soft_agg(x, kkidx_ref[...], G_KK_C,
                      akfw[...], akfb[...], akgw[...], akgb[...],
                      akhw[...], akhb[...])
    x = x + _soft_agg(x, ijidx_ref[...], G_IJ_C,
                      aifw[...], aifb[...], aigw[...], aigb[...],
                      aihw[...], aihb[...])
    x = _ln(x, l1g[...], l1b[...])
    x = _gr(x, g1gw[...], g1gb[...], g1r1w[...], g1r1b[...],
            g1r2w[...], g1r2b[...])
    x = _ln(x, l2g[...], l2b[...])
    x = _gr(x, g2gw[...], g2gb[...], g2r1w[...], g2r1b[...],
            g2r2w[...], g2r2b[...])
    out_net_ref[...] = x
    r = jax.nn.relu(x)
    dw = _dgT(r, wdw[...]) + bdw[...]
    lane = jax.lax.broadcasted_iota(i32, (N, 8), 1)
    out_dw_ref[...] = (jnp.where(lane < 2, dw, jax.nn.sigmoid(dw))
                       + ii_ref[...] * 1e-10)


def _sds(shape):
    return jax.ShapeDtypeStruct(shape, f32)


@jax.jit
def _run(net_t, inp_t, corr_t, ii_col, kk_row, kk_col, jj_row, jj_col,
         kkidx_col, ijidx_col, p, wdw, bdw):
    net_a = pl.pallas_call(
        _k1, out_shape=_sds((N, DIM)))(
        corr_t, net_t, inp_t, ii_col,
        p['corr_w1'], p['corr_b1'], p['corr_w2'], p['corr_b2'],
        p['corr_ln_g'], p['corr_ln_b'], p['corr_w3'], p['corr_b3'],
        p['norm_g'], p['norm_b'])

    net_b, jx = pl.pallas_call(
        _k2, out_shape=[_sds((N, DIM)),
                        jax.ShapeDtypeStruct((N, 1), i32)])(
        net_a, kk_row, kk_col, jj_row, jj_col,
        p['c1_w1'], p['c1_b1'], p['c1_w2'], p['c1_b2'])

    net_c = pl.pallas_call(
        _k_cmlp, out_shape=_sds((N, DIM)))(
        net_b, jx, p['c2_w1'], p['c2_b1'], p['c2_w2'], p['c2_b2'])

    net_f, dw = pl.pallas_call(
        _k3, out_shape=[_sds((N, DIM)), _sds((N, 8))])(
        net_c, kkidx_col, ijidx_col, ii_col,
        p['agg_kk_f_w'], p['agg_kk_f_b'], p['agg_kk_g_w'], p['agg_kk_g_b'],
        p['agg_kk_h_w'], p['agg_kk_h_b'],
        p['agg_ij_f_w'], p['agg_ij_f_b'], p['agg_ij_g_w'], p['agg_ij_g_b'],
        p['agg_ij_h_w'], p['agg_ij_h_b'],
        p['gru_ln1_g'], p['gru_ln1_b'],
        p['gr1_gate_w'], p['gr1_gate_b'], p['gr1_res_w1'], p['gr1_res_b1'],
        p['gr1_res_w2'], p['gr1_res_b2'],
        p['gru_ln2_g'], p['gru_ln2_b'],
        p['gr2_gate_w'], p['gr2_gate_b'], p['gr2_res_w1'], p['gr2_res_b1'],
        p['gr2_res_w2'], p['gr2_res_b2'],
        wdw, bdw)
    return net_f, dw


def kernel(net, inp, corr, flow, ii, jj, kk, kk_idx_map, G_kk, ij_idx_map,
           G_ij, params):
    del flow, G_kk, G_ij
    net_t = jnp.transpose(net[0, :, :, 0], (1, 0))
    inp_t = jnp.transpose(inp[0, :, :, 0], (1, 0))
    corr_t = jnp.transpose(corr[0, :, :, 0], (1, 0))
    ii_col = ii[0].astype(f32)
    jj_col = jj[0].astype(i32)
    kk_col = kk[0].astype(i32)
    jj_row = jj_col.reshape(1, N)
    kk_row = kk_col.reshape(1, N)
    kkidx_col = kk_idx_map.astype(i32).reshape(N, 1)
    ijidx_col = ij_idx_map.astype(i32).reshape(N, 1)

    p = {k: (v.reshape(1, -1) if v.ndim == 1 else v)
         for k, v in params.items()}
    wdw = jnp.concatenate(
        [params['d_w'], params['w_w'], jnp.zeros((4, DIM), f32)], axis=0)
    bdw = jnp.concatenate(
        [params['d_b'], params['w_b'], jnp.zeros((4,), f32)]).reshape(1, 8)

    net_f, dw = _run(net_t, inp_t, corr_t, ii_col, kk_row, kk_col, jj_row,
                     jj_col, kkidx_col, ijidx_col, p, wdw, bdw)
    return net_f[None], dw[None, :, 0:2], dw[None, :, 2:4]


# speedup vs baseline: 1.0026x; 1.0026x over previous
import jax
import jax.numpy as jnp
from jax.experimental import pallas as pl

DIM = 384
N = 4096
CORR_DIM = 882
G_KK_C = 512
G_IJ_C = 64
BLK = 256
NBLK = N // BLK

f32 = jnp.float32
i32 = jnp.int32


def _dgT(x, w):
    return jax.lax.dot_general(
        x, w, dimension_numbers=(((1,), (1,)), ((), ())),
        preferred_element_type=f32)


def _dg(x, w):
    return jax.lax.dot_general(
        x, w, dimension_numbers=(((1,), (0,)), ((), ())),
        preferred_element_type=f32)


def _dgTT(x, w):
    return jax.lax.dot_general(
        x, w, dimension_numbers=(((0,), (0,)), ((), ())),
        preferred_element_type=f32)


def _ln(x, g, b, eps=1e-3):
    mu = jnp.mean(x, axis=-1, keepdims=True)
    var = jnp.mean((x - mu) ** 2, axis=-1, keepdims=True)
    return (x - mu) / jnp.sqrt(var + eps) * g + b


def _k1(corr_ref, net_ref, inp_ref, ii_ref,
        w1, b1, w2, b2, lng, lnb, w3, b3, ng, nb, out_ref):
    c = jax.nn.relu(_dgT(corr_ref[...], w1[...]) + b1[...])
    c = _dgT(c, w2[...]) + b2[...]
    c = _ln(c, lng[...], lnb[...])
    c = jax.nn.relu(c)
    c = _dgT(c, w3[...]) + b3[...]
    ii_bias = jnp.sum(ii_ref[...]) * 1e-10
    x = net_ref[...] + inp_ref[...] + c + ii_bias
    out_ref[...] = _ln(x, ng[...], nb[...])


def _nb_tables(kk_row, kk_col, jj_row, jj_col):

    ohkk = (jax.lax.broadcasted_iota(i32, (N, G_KK_C), 1) == kk_col
            ).astype(f32)
    ohjj = (jax.lax.broadcasted_iota(i32, (N, G_IJ_C), 1) == jj_col
            ).astype(f32)

    iota_loc = jax.lax.broadcasted_iota(i32, (BLK, BLK), 1)
    row_loc = jax.lax.broadcasted_iota(i32, (BLK, BLK), 0)
    tri = iota_loc < row_loc
    p_run = jnp.zeros((G_KK_C, G_IJ_C), f32)
    firsts = []
    for b in range(NBLK):
        sl = slice(b * BLK, (b + 1) * BLK)
        okb = ohkk[sl]
        ojb = ohjj[sl]
        cnt_prior = jnp.sum(_dg(okb, p_run) * ojb, axis=1, keepdims=True)
        m2 = ((kk_row[:, sl] == kk_col[sl]) & (jj_row[:, sl] == jj_col[sl])
              & tri)
        cnt_within = jnp.sum(jnp.where(m2, 1.0, 0.0), axis=1, keepdims=True)
        firsts.append(cnt_prior + cnt_within < 0.5)
        p_run = p_run + _dgTT(okb, ojb)
    first = jnp.concatenate(firsts, axis=0).astype(f32)

    idx_i = jax.lax.broadcasted_iota(i32, (N, 1), 0)
    hi_f = (idx_i // G_IJ_C).astype(f32)
    lo_f = (idx_i % G_IJ_C).astype(f32)
    m_hi = _dgTT(ohkk * (hi_f * first), ohjj)
    m_lo = _dgTT(ohkk * (lo_f * first), ohjj)
    row_p = _dg(ohkk, p_run)
    row_hi = _dg(ohkk, m_hi)
    row_lo = _dg(ohkk, m_lo)

    iota64 = jax.lax.broadcasted_iota(i32, (N, G_IJ_C), 1)
    exists = row_p > 0.5
    vstar = jnp.max(jnp.where(exists & (iota64 < jj_col), iota64, -1),
                    axis=1, keepdims=True)
    vprime = jnp.min(jnp.where(exists & (iota64 > jj_col), iota64, G_IJ_C),
                     axis=1, keepdims=True)
    sel_i = (iota64 == vstar)
    ixv = (jnp.sum(jnp.where(sel_i, row_hi, 0.0), axis=1, keepdims=True)
           * G_IJ_C
           + jnp.sum(jnp.where(sel_i, row_lo, 0.0), axis=1, keepdims=True))
    sel_j = (iota64 == vprime)
    jxv = (jnp.sum(jnp.where(sel_j, row_hi, 0.0), axis=1, keepdims=True)
           * G_IJ_C
           + jnp.sum(jnp.where(sel_j, row_lo, 0.0), axis=1, keepdims=True))
    ix = jnp.where(vstar >= 1, ixv.astype(i32), 0)
    jx = jnp.where(vprime < G_IJ_C, jxv.astype(i32), 0)
    return ix, jx


def _gather_mlp(net_in, idx, w1, b1, w2, b2, blk=1024):
    iota = jax.lax.broadcasted_iota(i32, (blk, N), 1)
    blocks = []
    for b in range(N // blk):
        sl = slice(b * blk, (b + 1) * blk)
        oh = (iota == idx[sl]).astype(f32)
        gath = _dg(oh, net_in)
        h = jax.nn.relu(_dgT(gath, w1) + b1)
        upd = _dgT(h, w2) + b2
        blocks.append(net_in[sl] + upd)
    return jnp.concatenate(blocks, axis=0)


def _k2(net_ref, kk_row_ref, kk_col_ref, jj_row_ref, jj_col_ref,
        w1, b1, w2, b2, out_ref, jx_ref):
    ix, jx = _nb_tables(kk_row_ref[...], kk_col_ref[...],
                        jj_row_ref[...], jj_col_ref[...])
    jx_ref[...] = jx
    out_ref[...] = _gather_mlp(net_ref[...], ix,
                               w1[...], b1[...], w2[...], b2[...])


def _soft_agg(x, idx_col, G, fw, fb, gw, gb, hw, hb):
    fx = _dgT(x, fw) + fb
    gx = _dgT(x, gw) + gb
    gmax = jnp.max(gx, axis=0, keepdims=True)
    ex = jnp.exp(gx - gmax)
    oh = (jax.lax.broadcasted_iota(i32, (N, G), 1) == idx_col).astype(f32)
    esum = _dgTT(oh, ex)
    ynum = _dgTT(oh, fx * ex)
    y = ynum / jnp.where(esum > 0, esum, 1.0)
    hy = _dgT(y, hw) + hb
    return _dg(oh, hy)


def _gr(x, gw, gb, r1w, r1b, r2w, r2b):
    gate = jax.nn.sigmoid(_dgT(x, gw) + gb)
    res = _dgT(jax.nn.relu(_dgT(x, r1w) + r1b), r2w) + r2b
    return x + gate * res


def _k_cmlp(net_ref, idx_ref, w1, b1, w2, b2, out_ref):
    out_ref[...] = _gather_mlp(net_ref[...], idx_ref[...],
                               w1[...], b1[...], w2[...], b2[...])


def _k3(x_ref, kkidx_ref, ijidx_ref, ii_ref,
        akfw, akfb, akgw, akgb, akhw, akhb,
        aifw, aifb, aigw, aigb, aihw, aihb,
        l1g, l1b, g1gw, g1gb, g1r1w, g1r1b, g1r2w, g1r2b,
        l2g, l2b, g2gw, g2gb, g2r1w, g2r1b, g2r2w, g2r2b,
        wdw, bdw, out_net_ref, out_dw_ref):
    x = x_ref[...]
    x = x + _soft_agg(x, kkidx_ref[...], G_KK_C,
                      akfw[...], akfb[...], akgw[...], akgb[...],
                      akhw[...], akhb[...])
    x = x + _soft_agg(x, ijidx_ref[...], G_IJ_C,
                      aifw[...], aifb[...], aigw[...], aigb[...],
                      aihw[...], aihb[...])
    x = _ln(x, l1g[...], l1b[...])
    x = _gr(x, g1gw[...], g1gb[...], g1r1w[...], g1r1b[...],
            g1r2w[...], g1r2b[...])
    x = _ln(x, l2g[...], l2b[...])
    x = _gr(x, g2gw[...], g2gb[...], g2r1w[...], g2r1b[...],
            g2r2w[...], g2r2b[...])
    out_net_ref[...] = x
    r = jax.nn.relu(x)
    dw = _dgT(r, wdw[...]) + bdw[...]
    lane = jax.lax.broadcasted_iota(i32, (N, 8), 1)
    out_dw_ref[...] = (jnp.where(lane < 2, dw, jax.nn.sigmoid(dw))
                       + ii_ref[...] * 1e-10)


def _sds(shape):
    return jax.ShapeDtypeStruct(shape, f32)


@jax.jit
def _run(net_t, inp_t, corr_t, ii_col, kk_row, kk_col, jj_row, jj_col,
         kkidx_col, ijidx_col, p, wdw, bdw):
    net_a = pl.pallas_call(
        _k1, out_shape=_sds((N, DIM)))(
        corr_t, net_t, inp_t, ii_col,
        p['corr_w1'], p['corr_b1'], p['corr_w2'], p['corr_b2'],
        p['corr_ln_g'], p['corr_ln_b'], p['corr_w3'], p['corr_b3'],
        p['norm_g'], p['norm_b'])

    net_b, jx = pl.pallas_call(
        _k2, out_shape=[_sds((N, DIM)),
                        jax.ShapeDtypeStruct((N, 1), i32)])(
        net_a, kk_row, kk_col, jj_row, jj_col,
        p['c1_w1'], p['c1_b1'], p['c1_w2'], p['c1_b2'])

    net_c = pl.pallas_call(
        _k_cmlp, out_shape=_sds((N, DIM)))(
        net_b, jx, p['c2_w1'], p['c2_b1'], p['c2_w2'], p['c2_b2'])

    net_f, dw = pl.pallas_call(
        _k3, out_shape=[_sds((N, DIM)), _sds((N, 8))])(
        net_c, kkidx_col, ijidx_col, ii_col,
        p['agg_kk_f_w'], p['agg_kk_f_b'], p['agg_kk_g_w'], p['agg_kk_g_b'],
        p['agg_kk_h_w'], p['agg_kk_h_b'],
        p['agg_ij_f_w'], p['agg_ij_f_b'], p['agg_ij_g_w'], p['agg_ij_g_b'],
        p['agg_ij_h_w'], p['agg_ij_h_b'],
        p['gru_ln1_g'], p['gru_ln1_b'],
        p['gr1_gate_w'], p['gr1_gate_b'], p['gr1_res_w1'], p['gr1_res_b1'],
        p['gr1_res_w2'], p['gr1_res_b2'],
        p['gru_ln2_g'], p['gru_ln2_b'],
        p['gr2_gate_w'], p['gr2_gate_b'], p['gr2_res_w1'], p['gr2_res_b1'],
        p['gr2_res_w2'], p['gr2_res_b2'],
        wdw, bdw)
    return net_f, dw


def kernel(net, inp, corr, flow, ii, jj, kk, kk_idx_map, G_kk, ij_idx_map,
           G_ij, params):
    del flow, G_kk, G_ij
    net_t = jnp.transpose(net[0, :, :, 0], (1, 0))
    inp_t = jnp.transpose(inp[0, :, :, 0], (1, 0))
    corr_t = jnp.transpose(corr[0, :, :, 0], (1, 0))
    ii_col = ii[0].astype(f32)
    jj_col = jj[0].astype(i32)
    kk_col = kk[0].astype(i32)
    jj_row = jj_col.reshape(1, N)
    kk_row = kk_col.reshape(1, N)
    kkidx_col = kk_idx_map.astype(i32).reshape(N, 1)
    ijidx_col = ij_idx_map.astype(i32).reshape(N, 1)

    p = {k: (v.reshape(1, -1) if v.ndim == 1 else v)
         for k, v in params.items()}
    wdw = jnp.concatenate(
        [params['d_w'], params['w_w'], jnp.zeros((4, DIM), f32)], axis=0)
    bdw = jnp.concatenate(
        [params['d_b'], params['w_b'], jnp.zeros((4,), f32)]).reshape(1, 8)

    net_f, dw = _run(net_t, inp_t, corr_t, ii_col, kk_row, kk_col, jj_row,
                     jj_col, kkidx_col, ijidx_col, p, wdw, bdw)
    return net_f[None], dw[None, :, 0:2], dw[None, :, 2:4]
